# baseline (speedup 1.0000x reference)
import dataclasses
import functools

import jax
import jax.numpy as jnp
from jax import lax
from jax.experimental import pallas as pl
from jax.experimental.pallas import tpu as pltpu
from jax.experimental.pallas import tpu_sc as plsc

NC = 2
NS = 16
NW = NC * NS
L = 16


def _sc_aggregate(x, edges_flat, npred, n, d, e, p, e_per_w, k, rnd):
    p_pad = p + 128
    rows_per_sub = p_pad // NS
    p_per_w = p // NW
    p_per_sub = p // NS
    nrounds = e_per_w // rnd
    kept_cap = rnd + 3 * k + L

    zeros_agg = jnp.zeros((rows_per_sub, d), jnp.float32)

    mesh = plsc.VectorSubcoreMesh(core_axis_name="c", subcore_axis_name="s",
                                  num_cores=NC, num_subcores=NS)

    cp = pltpu.CompilerParams()
    if "needs_layout_passes" in pltpu.CompilerParams.__dataclass_fields__:
        cp = dataclasses.replace(cp, needs_layout_passes=False)

    @functools.partial(
        pl.kernel,
        compiler_params=cp,
        out_type=(
            jax.ShapeDtypeStruct((p, d), jnp.float32),
            jax.ShapeDtypeStruct((NC, p, d), jnp.float32),
            jax.ShapeDtypeStruct((NC * p,), jnp.float32),
        ),
        mesh=mesh,
        scratch_types=[
            pltpu.VMEM((n,), jnp.int32),
            pltpu.VMEM((p,), jnp.int32),
            pltpu.VMEM((rnd,), jnp.int32),
            pltpu.VMEM((rnd,), jnp.int32),
            pltpu.VMEM((rnd,), jnp.int32),
            pltpu.VMEM((rnd,), jnp.int32),
            pltpu.VMEM((kept_cap,), jnp.int32),
            pltpu.VMEM((kept_cap,), jnp.int32),
            pltpu.VMEM((1, k), jnp.int32),
            pltpu.VMEM((1, k), jnp.int32),
            pltpu.VMEM((k, d), jnp.float32),
            pltpu.VMEM((k, d), jnp.float32),
            pltpu.VMEM((p_pad,), jnp.float32),
            pltpu.VMEM((p_per_sub,), jnp.float32),
            pltpu.VMEM((p_per_sub,), jnp.float32),
            pltpu.VMEM((p,), jnp.float32),
            pltpu.VMEM((p_per_sub,), jnp.float32),
            pltpu.VMEM((p_per_sub,), jnp.int32),
            pltpu.VMEM_SHARED((p_pad, d), jnp.float32),
            pltpu.VMEM_SHARED((NS * p_pad,), jnp.float32),
            pltpu.VMEM_SHARED((p,), jnp.float32),
            pltpu.SemaphoreType.DMA,
            pltpu.SemaphoreType.DMA,
            pltpu.SemaphoreType.DMA,
            pltpu.SemaphoreType.DMA,
            pltpu.SemaphoreType.DMA,
            pltpu.SemaphoreType.DMA,
            pltpu.SemaphoreType.DMA,
        ],
    )
    def agg_kernel(x_hbm, edges_hbm, npred_hbm, zagg_hbm,
                   xg_hbm, oagg_hbm, odeg_hbm,
                   slot_tab, pidx_all, src_a, dst_a, src_b, dst_b,
                   kept_src, kept_slot, slot2d_a, slot2d_b, gbuf_a, gbuf_b,
                   deg_loc, deg_tmp, deg_acc, deg_all, deg_out, slot_idx_v,
                   aggc_sh, degs_sh, degf_sh,
                   ssa, sda, ssb, sdb, sga, sgb, sz):
        c = lax.axis_index("c")
        s = lax.axis_index("s")
        wid = s * NC + c
        ebase = wid * e_per_w

        pltpu.async_copy(
            zagg_hbm, aggc_sh.at[pl.ds(s * rows_per_sub, rows_per_sub)], sz)
        pltpu.sync_copy(npred_hbm, pidx_all)

        @pl.loop(0, p_pad // L)
        def _(i):
            deg_loc[pl.ds(i * L, L)] = jnp.zeros((L,), jnp.float32)

        @pl.loop(0, n // L)
        def _(i):
            slot_tab[pl.ds(i * L, L)] = jnp.full((L,), -1, jnp.int32)

        @pl.loop(0, p // L)
        def _(i):
            nv = pidx_all[pl.ds(i * L, L)]
            slots = lax.broadcasted_iota(jnp.int32, (L,), 0) + i * L
            plsc.store_scatter(slot_tab, [nv], slots)

        pltpu.make_async_copy(
            zagg_hbm, aggc_sh.at[pl.ds(s * rows_per_sub, rows_per_sub)],
            sz).wait()
        plsc.subcore_barrier()

        ones_v = jnp.zeros((L,), jnp.float32) + 1.0
        lane = lax.broadcasted_iota(jnp.int32, (L,), 0)

        @pl.loop(0, nrounds)
        def _(r):
            base = ebase + r * rnd
            pltpu.sync_copy(edges_hbm.at[pl.ds(base, rnd)], src_a)
            pltpu.sync_copy(edges_hbm.at[pl.ds(e + base, rnd)], dst_a)

            def filt(v, cnt):
                srcv = src_a[pl.ds(v * L, L)]
                dstv = dst_a[pl.ds(v * L, L)]
                slv = plsc.load_gather(slot_tab, [dstv])
                m = slv >= 0
                plsc.addupdate_scatter(deg_loc, [slv], ones_v, mask=m)
                plsc.store_compressed(kept_src.at[pl.ds(cnt, L)], srcv,
                                      mask=m)
                plsc.store_compressed(kept_slot.at[pl.ds(cnt, L)], slv,
                                      mask=m)
                return cnt + jnp.sum(m.astype(jnp.int32))

            cnt = lax.fori_loop(0, rnd // L, filt, jnp.int32(0))

            for i in range(3 * k // L):
                kept_src[pl.ds(cnt + i * L, L)] = lane * 8
                kept_slot[pl.ds(cnt + i * L, L)] = lane + p

            def issue(j, gbuf, sem):
                pltpu.async_copy(x_hbm.at[kept_src.at[pl.ds(j * k, k)]],
                                 gbuf, sem)

            def wait(j, gbuf, sem):
                pltpu.make_async_copy(
                    x_hbm.at[kept_src.at[pl.ds(j * k, k)]], gbuf, sem).wait()

            def scat(j, slot2d, gbuf):
                for i in range(k // L):
                    slot2d[0, pl.ds(i * L, L)] = \
                        kept_slot[pl.ds(j * k + i * L, L)]
                pltpu.sync_copy(gbuf, aggc_sh.at[slot2d.at[0]], add=True)

            nch = (cnt + (k - 1)) // k
            nch_run = nch + (nch & 1)

            def pair(jj, _):
                j = jj * 2
                issue(j + 1, gbuf_b, sgb)
                wait(j, gbuf_a, sga)
                scat(j, slot2d_a, gbuf_a)
                issue(j + 2, gbuf_a, sga)
                wait(j + 1, gbuf_b, sgb)
                scat(j + 1, slot2d_b, gbuf_b)
                return 0

            @pl.when(nch_run > 0)
            def _():
                issue(0, gbuf_a, sga)
                lax.fori_loop(0, nch_run // 2, pair, jnp.int32(0))
                wait(nch_run, gbuf_a, sga)

        pltpu.sync_copy(deg_loc, degs_sh.at[pl.ds(s * p_pad, p_pad)])

        pltpu.sync_copy(x_hbm.at[pidx_all.at[pl.ds(wid * p_per_w, p_per_w)]],
                        gbuf_a.at[0:p_per_w])
        pltpu.sync_copy(gbuf_a.at[0:p_per_w],
                        xg_hbm.at[pl.ds(wid * p_per_w, p_per_w)])

        plsc.subcore_barrier()

        @pl.loop(0, p_per_sub // L)
        def _(i):
            deg_acc[pl.ds(i * L, L)] = jnp.zeros((L,), jnp.float32)

        @pl.loop(0, NS)
        def _(t):
            pltpu.sync_copy(
                degs_sh.at[pl.ds(t * p_pad + s * p_per_sub, p_per_sub)],
                deg_tmp)
            for i in range(p_per_sub // L):
                plsc.addupdate(deg_acc.at[pl.ds(i * L, L)],
                               deg_tmp[pl.ds(i * L, L)])

        pltpu.sync_copy(deg_acc, degf_sh.at[pl.ds(s * p_per_sub, p_per_sub)])

        plsc.subcore_barrier()

        @pl.loop(0, p_per_sub // L)
        def _(i):
            nv = pidx_all[pl.ds(s * p_per_sub + i * L, L)]
            slot_idx_v[pl.ds(i * L, L)] = plsc.load_gather(slot_tab, [nv])

        pltpu.sync_copy(degf_sh, deg_all)

        @pl.loop(0, p_per_sub // L)
        def _(i):
            slv = slot_idx_v[pl.ds(i * L, L)]
            deg_out[pl.ds(i * L, L)] = plsc.load_gather(deg_all, [slv])

        pltpu.sync_copy(aggc_sh.at[slot_idx_v], gbuf_a)
        pltpu.sync_copy(gbuf_a,
                        oagg_hbm.at[c, pl.ds(s * p_per_sub, p_per_sub)])
        pltpu.sync_copy(
            deg_out, odeg_hbm.at[pl.ds(c * p + s * p_per_sub, p_per_sub)])

    return agg_kernel(x, edges_flat, npred, zeros_agg)


def _tc_head(xg, agg2, deg2, W_self, W_neigh, b2, Wp_pad, bp_pad, p, d):
    def body(xg_ref, agg_ref, deg_ref, ws_ref, wn_ref, b_ref, wp_ref, bp_ref,
             o_ref):
        agg = agg_ref[0] + agg_ref[1]
        dcol = deg_ref[0] + deg_ref[1]
        mean = agg / jnp.maximum(dcol, 1.0)
        h = (
            jnp.dot(xg_ref[...], ws_ref[...],
                    preferred_element_type=jnp.float32)
            + jnp.dot(mean, wn_ref[...],
                      preferred_element_type=jnp.float32)
            + b_ref[...]
        )
        n2 = jnp.sum(h * h, axis=1, keepdims=True)
        z = h * lax.rsqrt(jnp.maximum(n2, 1e-24))
        o_ref[...] = (
            jnp.dot(z, wp_ref[...], preferred_element_type=jnp.float32)
            + bp_ref[...]
        )

    return pl.pallas_call(
        body,
        out_shape=jax.ShapeDtypeStruct((p, d), jnp.float32),
    )(xg, agg2, deg2, W_self, W_neigh, b2, Wp_pad, bp_pad)


def kernel(x, edge_index, nodes_to_predict, W_self, W_neigh, b, W_pred,
           b_pred):
    n, d = x.shape
    e = edge_index.shape[1]
    p = nodes_to_predict.shape[0]
    c_out = W_pred.shape[1]

    e_per_w = e // NW
    k = 128
    rnd = 2000

    xg, agg2, deg2 = _sc_aggregate(x, edge_index.reshape(2 * e),
                                   nodes_to_predict, n, d, e, p, e_per_w, k,
                                   rnd)

    b2 = b.reshape(1, d)
    Wp_pad = jnp.pad(W_pred, ((0, 0), (0, d - c_out)))
    bp_pad = jnp.pad(b_pred, (0, d - c_out)).reshape(1, d)

    y_full = _tc_head(xg, agg2, deg2.reshape(NC, p, 1), W_self, W_neigh, b2,
                      Wp_pad, bp_pad, p, d)
    return y_full[:, :c_out]


# speedup vs baseline: 1.5367x; 1.5367x over previous
import dataclasses
import functools

import jax
import jax.numpy as jnp
from jax import lax
from jax.experimental import pallas as pl
from jax.experimental.pallas import tpu as pltpu
from jax.experimental.pallas import tpu_sc as plsc

NC = 2
NS = 16
NW = NC * NS
L = 16


def _sc_aggregate(x, edges_flat, npred, n, d, e, p, e_per_w, k, rnd):
    p_pad = p + 128
    rows_per_sub = p_pad // NS
    p_per_w = p // NW
    p_per_sub = p // NS
    nrounds = e_per_w // rnd
    kept_cap = rnd + 3 * k + L

    zeros_agg = jnp.zeros((rows_per_sub, d), jnp.float32)

    mesh = plsc.VectorSubcoreMesh(core_axis_name="c", subcore_axis_name="s",
                                  num_cores=NC, num_subcores=NS)

    cp = pltpu.CompilerParams()
    if "needs_layout_passes" in pltpu.CompilerParams.__dataclass_fields__:
        cp = dataclasses.replace(cp, needs_layout_passes=False)

    @functools.partial(
        pl.kernel,
        compiler_params=cp,
        out_type=(
            jax.ShapeDtypeStruct((p, d), jnp.float32),
            jax.ShapeDtypeStruct((NC, p, d), jnp.float32),
            jax.ShapeDtypeStruct((NC * p,), jnp.float32),
        ),
        mesh=mesh,
        scratch_types=[
            pltpu.VMEM((n,), jnp.int32),
            pltpu.VMEM((p,), jnp.int32),
            pltpu.VMEM((rnd,), jnp.int32),
            pltpu.VMEM((rnd,), jnp.int32),
            pltpu.VMEM((kept_cap,), jnp.int32),
            pltpu.VMEM((kept_cap,), jnp.int32),
            pltpu.VMEM((1, k), jnp.int32),
            pltpu.VMEM((2 * k, d), jnp.float32),
            pltpu.VMEM((p_pad,), jnp.float32),
            pltpu.VMEM((p_per_sub,), jnp.float32),
            pltpu.VMEM((p_per_sub,), jnp.float32),
            pltpu.VMEM((p,), jnp.float32),
            pltpu.VMEM((p_per_sub,), jnp.float32),
            pltpu.VMEM((p_per_sub,), jnp.int32),
            pltpu.VMEM_SHARED((p_pad, d), jnp.float32),
            pltpu.VMEM_SHARED((NS * p_pad,), jnp.float32),
            pltpu.VMEM_SHARED((p,), jnp.float32),
            pltpu.SemaphoreType.DMA,
        ],
    )
    def agg_kernel(x_hbm, edges_hbm, npred_hbm, zagg_hbm,
                   xg_hbm, oagg_hbm, odeg_hbm,
                   slot_tab, pidx_all, src_a, dst_a,
                   kept_src, kept_slot, slot2d_a, gbuf_a,
                   deg_loc, deg_tmp, deg_acc, deg_all, deg_out, slot_idx_v,
                   aggc_sh, degs_sh, degf_sh, sz):
        c = lax.axis_index("c")
        s = lax.axis_index("s")
        wid = s * NC + c
        ebase = wid * e_per_w

        pltpu.async_copy(
            zagg_hbm, aggc_sh.at[pl.ds(s * rows_per_sub, rows_per_sub)], sz)
        pltpu.sync_copy(npred_hbm, pidx_all)

        @pl.loop(0, p_pad // L)
        def _(i):
            deg_loc[pl.ds(i * L, L)] = jnp.zeros((L,), jnp.float32)

        @pl.loop(0, n // L)
        def _(i):
            slot_tab[pl.ds(i * L, L)] = jnp.full((L,), -1, jnp.int32)

        @pl.loop(0, p // L)
        def _(i):
            nv = pidx_all[pl.ds(i * L, L)]
            slots = lax.broadcasted_iota(jnp.int32, (L,), 0) + i * L
            plsc.store_scatter(slot_tab, [nv], slots)

        pltpu.make_async_copy(
            zagg_hbm, aggc_sh.at[pl.ds(s * rows_per_sub, rows_per_sub)],
            sz).wait()
        plsc.subcore_barrier()

        ones_v = jnp.zeros((L,), jnp.float32) + 1.0
        lane = lax.broadcasted_iota(jnp.int32, (L,), 0)

        @pl.loop(0, nrounds)
        def _(r):
            base = ebase + r * rnd
            pltpu.sync_copy(edges_hbm.at[pl.ds(base, rnd)], src_a)
            pltpu.sync_copy(edges_hbm.at[pl.ds(e + base, rnd)], dst_a)

            def filt(v, cnt):
                srcv = src_a[pl.ds(v * L, L)]
                dstv = dst_a[pl.ds(v * L, L)]
                slv = plsc.load_gather(slot_tab, [dstv])
                m = slv >= 0
                plsc.addupdate_scatter(deg_loc, [slv], ones_v, mask=m)
                plsc.store_compressed(kept_src.at[pl.ds(cnt, L)], srcv,
                                      mask=m)
                plsc.store_compressed(kept_slot.at[pl.ds(cnt, L)], slv,
                                      mask=m)
                return cnt + jnp.sum(m.astype(jnp.int32))

            cnt = lax.fori_loop(0, rnd // L, filt, jnp.int32(0))

            for i in range(2 * k // L):
                kept_src[pl.ds(cnt + i * L, L)] = lane * 8
                kept_slot[pl.ds(cnt + i * L, L)] = lane + p

            def chunk(j, _):
                pltpu.sync_copy(
                    x_hbm.at[kept_src.at[pl.ds(j * (2 * k), 2 * k)]], gbuf_a)
                for h in range(2):
                    for i in range(k // L):
                        slot2d_a[0, pl.ds(i * L, L)] = kept_slot[
                            pl.ds(j * (2 * k) + h * k + i * L, L)]
                    pltpu.sync_copy(gbuf_a.at[h * k:(h + 1) * k],
                                    aggc_sh.at[slot2d_a.at[0]], add=True)
                return 0

            nch = (cnt + (2 * k - 1)) // (2 * k)
            lax.fori_loop(0, nch, chunk, jnp.int32(0))

        pltpu.sync_copy(deg_loc, degs_sh.at[pl.ds(s * p_pad, p_pad)])

        pltpu.sync_copy(x_hbm.at[pidx_all.at[pl.ds(wid * p_per_w, p_per_w)]],
                        gbuf_a.at[0:p_per_w])
        pltpu.sync_copy(gbuf_a.at[0:p_per_w],
                        xg_hbm.at[pl.ds(wid * p_per_w, p_per_w)])

        plsc.subcore_barrier()

        @pl.loop(0, p_per_sub // L)
        def _(i):
            deg_acc[pl.ds(i * L, L)] = jnp.zeros((L,), jnp.float32)

        @pl.loop(0, NS)
        def _(t):
            pltpu.sync_copy(
                degs_sh.at[pl.ds(t * p_pad + s * p_per_sub, p_per_sub)],
                deg_tmp)
            for i in range(p_per_sub // L):
                plsc.addupdate(deg_acc.at[pl.ds(i * L, L)],
                               deg_tmp[pl.ds(i * L, L)])

        pltpu.sync_copy(deg_acc, degf_sh.at[pl.ds(s * p_per_sub, p_per_sub)])

        plsc.subcore_barrier()

        @pl.loop(0, p_per_sub // L)
        def _(i):
            nv = pidx_all[pl.ds(s * p_per_sub + i * L, L)]
            slot_idx_v[pl.ds(i * L, L)] = plsc.load_gather(slot_tab, [nv])

        pltpu.sync_copy(degf_sh, deg_all)

        @pl.loop(0, p_per_sub // L)
        def _(i):
            slv = slot_idx_v[pl.ds(i * L, L)]
            deg_out[pl.ds(i * L, L)] = plsc.load_gather(deg_all, [slv])

        pltpu.sync_copy(aggc_sh.at[slot_idx_v], gbuf_a.at[0:p_per_sub])
        pltpu.sync_copy(gbuf_a.at[0:p_per_sub],
                        oagg_hbm.at[c, pl.ds(s * p_per_sub, p_per_sub)])
        pltpu.sync_copy(
            deg_out, odeg_hbm.at[pl.ds(c * p + s * p_per_sub, p_per_sub)])

    return agg_kernel(x, edges_flat, npred, zeros_agg)


def _tc_head(xg, agg2, deg2, W_self, W_neigh, b2, Wp_pad, bp_pad, p, d):
    def body(xg_ref, agg_ref, deg_ref, ws_ref, wn_ref, b_ref, wp_ref, bp_ref,
             o_ref):
        agg = agg_ref[0] + agg_ref[1]
        dcol = deg_ref[0] + deg_ref[1]
        mean = agg / jnp.maximum(dcol, 1.0)
        h = (
            jnp.dot(xg_ref[...], ws_ref[...],
                    preferred_element_type=jnp.float32)
            + jnp.dot(mean, wn_ref[...],
                      preferred_element_type=jnp.float32)
            + b_ref[...]
        )
        n2 = jnp.sum(h * h, axis=1, keepdims=True)
        z = h * lax.rsqrt(jnp.maximum(n2, 1e-24))
        o_ref[...] = (
            jnp.dot(z, wp_ref[...], preferred_element_type=jnp.float32)
            + bp_ref[...]
        )

    return pl.pallas_call(
        body,
        out_shape=jax.ShapeDtypeStruct((p, d), jnp.float32),
    )(xg, agg2, deg2, W_self, W_neigh, b2, Wp_pad, bp_pad)


def kernel(x, edge_index, nodes_to_predict, W_self, W_neigh, b, W_pred,
           b_pred):
    n, d = x.shape
    e = edge_index.shape[1]
    p = nodes_to_predict.shape[0]
    c_out = W_pred.shape[1]

    e_per_w = e // NW
    k = 128
    rnd = 2000

    xg, agg2, deg2 = _sc_aggregate(x, edge_index.reshape(2 * e),
                                   nodes_to_predict, n, d, e, p, e_per_w, k,
                                   rnd)

    b2 = b.reshape(1, d)
    Wp_pad = jnp.pad(W_pred, ((0, 0), (0, d - c_out)))
    bp_pad = jnp.pad(b_pred, (0, d - c_out)).reshape(1, d)

    y_full = _tc_head(xg, agg2, deg2.reshape(NC, p, 1), W_self, W_neigh, b2,
                      Wp_pad, bp_pad, p, d)
    return y_full[:, :c_out]


# speedup vs baseline: 2.4807x; 1.6143x over previous
import dataclasses
import functools

import jax
import jax.numpy as jnp
from jax import lax
from jax.experimental import pallas as pl
from jax.experimental.pallas import tpu as pltpu
from jax.experimental.pallas import tpu_sc as plsc

NC = 2
NS = 16
NW = NC * NS
L = 16


def _sc_aggregate(x, edges_flat, npred, n, d, e, p, e_per_w, k, rnd):
    p_pad = p + 128
    rows_per_sub = p_pad // NS
    p_per_w = p // NW
    p_per_sub = p // NS
    nrounds = e_per_w // rnd
    kept_cap = rnd + 3 * k + L

    zeros_agg = jnp.zeros((rows_per_sub, d), jnp.float32)

    mesh = plsc.VectorSubcoreMesh(core_axis_name="c", subcore_axis_name="s",
                                  num_cores=NC, num_subcores=NS)

    cp = pltpu.CompilerParams()
    if "needs_layout_passes" in pltpu.CompilerParams.__dataclass_fields__:
        cp = dataclasses.replace(cp, needs_layout_passes=False)

    @functools.partial(
        pl.kernel,
        compiler_params=cp,
        out_type=(
            jax.ShapeDtypeStruct((p, d), jnp.float32),
            jax.ShapeDtypeStruct((NC, p, d), jnp.float32),
            jax.ShapeDtypeStruct((NC * p,), jnp.float32),
        ),
        mesh=mesh,
        scratch_types=[
            pltpu.VMEM((n,), jnp.int32),
            pltpu.VMEM((p,), jnp.int32),
            pltpu.VMEM((rnd,), jnp.int32),
            pltpu.VMEM((rnd,), jnp.int32),
            pltpu.VMEM((kept_cap,), jnp.int32),
            pltpu.VMEM((kept_cap,), jnp.int32),
            pltpu.VMEM((1, k), jnp.int32),
            pltpu.VMEM((2 * k, d), jnp.float32),
            pltpu.VMEM((p_pad,), jnp.float32),
            pltpu.VMEM((p_per_sub,), jnp.float32),
            pltpu.VMEM((p_per_sub,), jnp.float32),
            pltpu.VMEM((p,), jnp.float32),
            pltpu.VMEM((p_per_sub,), jnp.float32),
            pltpu.VMEM((p_per_sub,), jnp.int32),
            pltpu.VMEM_SHARED((p_pad, d), jnp.float32),
            pltpu.VMEM_SHARED((NS * p_pad,), jnp.float32),
            pltpu.VMEM_SHARED((p,), jnp.float32),
            pltpu.SemaphoreType.DMA,
        ],
    )
    def agg_kernel(x_hbm, edges_hbm, npred_hbm, zagg_hbm,
                   xg_hbm, oagg_hbm, odeg_hbm,
                   slot_tab, pidx_all, src_a, dst_a,
                   kept_src, kept_slot, slot2d_a, gbuf_a,
                   deg_loc, deg_tmp, deg_acc, deg_all, deg_out, slot_idx_v,
                   aggc_sh, degs_sh, degf_sh, sz):
        c = lax.axis_index("c")
        s = lax.axis_index("s")
        wid = s * NC + c
        ebase = wid * e_per_w

        pltpu.async_copy(
            zagg_hbm, aggc_sh.at[pl.ds(s * rows_per_sub, rows_per_sub)], sz)
        pltpu.sync_copy(npred_hbm, pidx_all)

        @pl.loop(0, p_pad // L)
        def _(i):
            deg_loc[pl.ds(i * L, L)] = jnp.zeros((L,), jnp.float32)

        @pl.loop(0, n // L)
        def _(i):
            slot_tab[pl.ds(i * L, L)] = jnp.full((L,), -1, jnp.int32)

        @pl.loop(0, p // L)
        def _(i):
            nv = pidx_all[pl.ds(i * L, L)]
            slots = lax.broadcasted_iota(jnp.int32, (L,), 0) + i * L
            plsc.store_scatter(slot_tab, [nv], slots)

        pltpu.make_async_copy(
            zagg_hbm, aggc_sh.at[pl.ds(s * rows_per_sub, rows_per_sub)],
            sz).wait()
        plsc.subcore_barrier()

        ones_v = jnp.zeros((L,), jnp.float32) + 1.0
        lane = lax.broadcasted_iota(jnp.int32, (L,), 0)

        @pl.loop(0, nrounds)
        def _(r):
            base = ebase + r * rnd
            pltpu.sync_copy(edges_hbm.at[pl.ds(base, rnd)], src_a)
            pltpu.sync_copy(edges_hbm.at[pl.ds(e + base, rnd)], dst_a)

            def filt(v, cnt):
                srcv = src_a[pl.ds(v * L, L)]
                dstv = dst_a[pl.ds(v * L, L)]
                slv = plsc.load_gather(slot_tab, [dstv])
                m = slv >= 0
                plsc.addupdate_scatter(deg_loc, [slv], ones_v, mask=m)
                plsc.store_compressed(kept_src.at[pl.ds(cnt, L)], srcv,
                                      mask=m)
                plsc.store_compressed(kept_slot.at[pl.ds(cnt, L)], slv,
                                      mask=m)
                return cnt + jnp.sum(m.astype(jnp.int32))

            cnt = lax.fori_loop(0, rnd // L, filt, jnp.int32(0))

            for i in range(k // L):
                kept_src[pl.ds(cnt + i * L, L)] = lane * 8
                kept_slot[pl.ds(cnt + i * L, L)] = lane + p

            def chunk(j, _):
                for i in range(k // L):
                    slot2d_a[0, pl.ds(i * L, L)] = \
                        kept_slot[pl.ds(j * k + i * L, L)]
                pltpu.sync_copy(x_hbm.at[kept_src.at[pl.ds(j * k, k)]],
                                gbuf_a.at[0:k])
                pltpu.sync_copy(gbuf_a.at[0:k], aggc_sh.at[slot2d_a.at[0]],
                                add=True)
                return 0

            nch = (cnt + (k - 1)) // k
            lax.fori_loop(0, nch, chunk, jnp.int32(0))

        pltpu.sync_copy(deg_loc, degs_sh.at[pl.ds(s * p_pad, p_pad)])

        pltpu.sync_copy(x_hbm.at[pidx_all.at[pl.ds(wid * p_per_w, p_per_w)]],
                        gbuf_a.at[0:p_per_w])
        pltpu.sync_copy(gbuf_a.at[0:p_per_w],
                        xg_hbm.at[pl.ds(wid * p_per_w, p_per_w)])

        plsc.subcore_barrier()

        @pl.loop(0, p_per_sub // L)
        def _(i):
            deg_acc[pl.ds(i * L, L)] = jnp.zeros((L,), jnp.float32)

        @pl.loop(0, NS)
        def _(t):
            pltpu.sync_copy(
                degs_sh.at[pl.ds(t * p_pad + s * p_per_sub, p_per_sub)],
                deg_tmp)
            for i in range(p_per_sub // L):
                plsc.addupdate(deg_acc.at[pl.ds(i * L, L)],
                               deg_tmp[pl.ds(i * L, L)])

        pltpu.sync_copy(deg_acc, degf_sh.at[pl.ds(s * p_per_sub, p_per_sub)])

        plsc.subcore_barrier()

        @pl.loop(0, p_per_sub // L)
        def _(i):
            nv = pidx_all[pl.ds(s * p_per_sub + i * L, L)]
            slot_idx_v[pl.ds(i * L, L)] = plsc.load_gather(slot_tab, [nv])

        pltpu.sync_copy(degf_sh, deg_all)

        @pl.loop(0, p_per_sub // L)
        def _(i):
            slv = slot_idx_v[pl.ds(i * L, L)]
            deg_out[pl.ds(i * L, L)] = plsc.load_gather(deg_all, [slv])

        pltpu.sync_copy(aggc_sh.at[slot_idx_v], gbuf_a.at[0:p_per_sub])
        pltpu.sync_copy(gbuf_a.at[0:p_per_sub],
                        oagg_hbm.at[c, pl.ds(s * p_per_sub, p_per_sub)])
        pltpu.sync_copy(
            deg_out, odeg_hbm.at[pl.ds(c * p + s * p_per_sub, p_per_sub)])

    return agg_kernel(x, edges_flat, npred, zeros_agg)


def _tc_head(xg, agg2, deg2, W_self, W_neigh, b2, Wp_pad, bp_pad, p, d):
    def body(xg_ref, agg_ref, deg_ref, ws_ref, wn_ref, b_ref, wp_ref, bp_ref,
             o_ref):
        agg = agg_ref[0] + agg_ref[1]
        dcol = deg_ref[0] + deg_ref[1]
        mean = agg / jnp.maximum(dcol, 1.0)
        h = (
            jnp.dot(xg_ref[...], ws_ref[...],
                    preferred_element_type=jnp.float32)
            + jnp.dot(mean, wn_ref[...],
                      preferred_element_type=jnp.float32)
            + b_ref[...]
        )
        n2 = jnp.sum(h * h, axis=1, keepdims=True)
        z = h * lax.rsqrt(jnp.maximum(n2, 1e-24))
        o_ref[...] = (
            jnp.dot(z, wp_ref[...], preferred_element_type=jnp.float32)
            + bp_ref[...]
        )

    return pl.pallas_call(
        body,
        out_shape=jax.ShapeDtypeStruct((p, d), jnp.float32),
    )(xg, agg2, deg2, W_self, W_neigh, b2, Wp_pad, bp_pad)


def kernel(x, edge_index, nodes_to_predict, W_self, W_neigh, b, W_pred,
           b_pred):
    n, d = x.shape
    e = edge_index.shape[1]
    p = nodes_to_predict.shape[0]
    c_out = W_pred.shape[1]

    e_per_w = e // NW
    k = 128
    rnd = 2000

    xg, agg2, deg2 = _sc_aggregate(x, edge_index.reshape(2 * e),
                                   nodes_to_predict, n, d, e, p, e_per_w, k,
                                   rnd)

    b2 = b.reshape(1, d)
    Wp_pad = jnp.pad(W_pred, ((0, 0), (0, d - c_out)))
    bp_pad = jnp.pad(b_pred, (0, d - c_out)).reshape(1, d)

    y_full = _tc_head(xg, agg2, deg2.reshape(NC, p, 1), W_self, W_neigh, b2,
                      Wp_pad, bp_pad, p, d)
    return y_full[:, :c_out]
